# baseline (speedup 1.0000x reference)
import functools

import jax
import jax.numpy as jnp
from jax import lax
from jax.experimental import pallas as pl
from jax.experimental.pallas import tpu as pltpu
from jax.experimental.pallas import tpu_sc as plsc

N = 10000
E = 640000
B = 64
ROW_BLK = 1000

NTILES = 16
K = 50
SUP = 40
EROWS = E // K
ZR = 624
TAIL = N - NTILES * ZR
DH = 128
RING = 5
LOOK = 2


def _prop_body(edge_split, tab, src_a, src_b, dst2d, zrows, out,
               acc, sbuf, dbuf, gbuf, gsem, ssem):
    cid = lax.axis_index("c")
    sid = lax.axis_index("s")

    pltpu.sync_copy(zrows, acc.at[pl.ds(sid * ZR, ZR)])

    @pl.when(sid == NTILES - 1)
    def _():
        pltpu.sync_copy(zrows.at[pl.ds(0, TAIL)],
                        acc.at[pl.ds(NTILES * ZR, TAIL)])

    plsc.subcore_barrier()

    if edge_split:
        cpt = EROWS // (2 * NTILES)
        row0 = (cid * NTILES + sid) * cpt
    else:
        cpt = EROWS // NTILES
        row0 = sid * cpt
    sups = cpt // SUP

    def sup_body(s, _):
        r0 = row0 + s * SUP

        @pl.when(cid == 0)
        def _():
            pltpu.sync_copy(src_a.at[pl.ds(r0, SUP)], sbuf)

        @pl.when(cid == 1)
        def _():
            pltpu.sync_copy(src_b.at[pl.ds(r0, SUP)], sbuf)

        pltpu.sync_copy(dst2d.at[pl.ds(r0, SUP)], dbuf)

        hg = [None] * RING
        hs = [None] * RING
        for j in range(LOOK):
            hg[j] = pltpu.async_copy(tab.at[sbuf.at[j]], gbuf.at[j],
                                     gsem.at[j])
        for j in range(SUP):
            r = j % RING
            jn = j + LOOK
            if jn < SUP:
                rn = jn % RING
                if hs[rn] is not None:
                    hs[rn].wait()
                hg[rn] = pltpu.async_copy(tab.at[sbuf.at[jn]], gbuf.at[rn],
                                          gsem.at[rn])
            hg[r].wait()
            hs[r] = pltpu.async_copy(gbuf.at[r], acc.at[dbuf.at[j]],
                                     ssem.at[r], add=True)
        for r in range(RING):
            if hs[r] is not None:
                hs[r].wait()
        return 0

    lax.fori_loop(0, sups, sup_body, 0)

    plsc.subcore_barrier()
    rows = pl.ds(sid * ZR, ZR)
    tail = pl.ds(NTILES * ZR, TAIL)

    @pl.when(cid == 0)
    def _():
        pltpu.sync_copy(acc.at[rows], out.at[0].at[rows])

        @pl.when(sid == NTILES - 1)
        def _():
            pltpu.sync_copy(acc.at[tail], out.at[0].at[tail])

    @pl.when(cid == 1)
    def _():
        pltpu.sync_copy(acc.at[rows], out.at[1].at[rows])

        @pl.when(sid == NTILES - 1)
        def _():
            pltpu.sync_copy(acc.at[tail], out.at[1].at[tail])


def _prop_sc(tab, src_a, src_b, dst2d, zrows, edge_split):
    mesh = plsc.VectorSubcoreMesh(core_axis_name="c", subcore_axis_name="s",
                                  num_cores=2, num_subcores=NTILES)
    return pl.kernel(
        functools.partial(_prop_body, edge_split),
        out_type=jax.ShapeDtypeStruct((2, N, DH), jnp.float32),
        mesh=mesh,
        scratch_types=[
            pltpu.VMEM_SHARED((N, DH), jnp.float32),
            pltpu.VMEM((SUP, K), jnp.int32),
            pltpu.VMEM((SUP, K), jnp.int32),
            pltpu.VMEM((RING, K, DH), jnp.float32),
            pltpu.SemaphoreType.DMA((RING,)),
            pltpu.SemaphoreType.DMA((RING,)),
        ],
    )(tab, src_a, src_b, dst2d, zrows)


def _pre_body(deg_ref, x_ref, dinv_ref, y_ref):
    deg = deg_ref[...]
    dinv = lax.rsqrt(deg + 1.0)
    dinv_ref[...] = dinv
    y = dinv * x_ref[...]
    y_ref[...] = jnp.concatenate(
        [y, jnp.zeros((y.shape[0], DH - 90), jnp.float32)], axis=1)


def _pre_call(deg, x):
    grid = (N // ROW_BLK,)
    return pl.pallas_call(
        _pre_body,
        grid=grid,
        in_specs=[
            pl.BlockSpec((ROW_BLK, 1), lambda i: (i, 0)),
            pl.BlockSpec((ROW_BLK, 90), lambda i: (i, 0)),
        ],
        out_specs=[
            pl.BlockSpec((ROW_BLK, 1), lambda i: (i, 0)),
            pl.BlockSpec((ROW_BLK, DH), lambda i: (i, 0)),
        ],
        out_shape=[
            jax.ShapeDtypeStruct((N, 1), jnp.float32),
            jax.ShapeDtypeStruct((N, DH), jnp.float32),
        ],
    )(deg, x)


def _layer_body(s_ref, y_ref, dinv_ref, w_ref, b_ref, out_ref, *,
                s_mode, y_split, out_mode):
    dinv = dinv_ref[...]
    if s_mode == "sum":
        s = s_ref[0] + s_ref[1]
    else:
        s = jnp.concatenate([s_ref[0], s_ref[1]], axis=1)
    if y_split:
        y = jnp.concatenate([y_ref[0], y_ref[1]], axis=1)
    else:
        y = y_ref[...]
    t = dinv * (s + y)
    h = jnp.maximum(
        jnp.dot(t, w_ref[...], preferred_element_type=jnp.float32)
        + b_ref[...],
        0.0,
    )
    if out_mode == "plain":
        out_ref[...] = h
    elif out_mode == "flat":
        yn = dinv * h
        out_ref[...] = yn
    else:
        yn = dinv * h
        out_ref[0] = yn[:, :DH]
        out_ref[1] = yn[:, DH:]


def _layer_call(s, y, dinv, w, b, *, s_mode, y_split, out_mode):
    din, dout = w.shape
    grid = (N // ROW_BLK,)
    if y_split:
        y_spec = pl.BlockSpec((2, ROW_BLK, DH), lambda i: (0, i, 0))
    else:
        y_spec = pl.BlockSpec((ROW_BLK, din), lambda i: (i, 0))
    if out_mode == "split":
        out_spec = pl.BlockSpec((2, ROW_BLK, DH), lambda i: (0, i, 0))
        out_shape = jax.ShapeDtypeStruct((2, N, DH), jnp.float32)
    else:
        out_spec = pl.BlockSpec((ROW_BLK, dout), lambda i: (i, 0))
        out_shape = jax.ShapeDtypeStruct((N, dout), jnp.float32)
    return pl.pallas_call(
        functools.partial(_layer_body, s_mode=s_mode, y_split=y_split,
                          out_mode=out_mode),
        grid=grid,
        in_specs=[
            pl.BlockSpec((2, ROW_BLK, DH), lambda i: (0, i, 0)),
            y_spec,
            pl.BlockSpec((ROW_BLK, 1), lambda i: (i, 0)),
            pl.BlockSpec((din, dout), lambda i: (0, 0)),
            pl.BlockSpec((1, dout), lambda i: (0, 0)),
        ],
        out_specs=out_spec,
        out_shape=out_shape,
    )(s, y, dinv, w, b.reshape(1, dout))


def _pool_head_body(h_ref, batch_ref, fw1_ref, fb1_ref, fw2_ref, fb2_ref,
                    fw3_ref, fb3_ref, out_ref, macc, sacc, cacc):
    i = pl.program_id(0)
    nsteps = pl.num_programs(0)

    @pl.when(i == 0)
    def _():
        macc[...] = jnp.full((B, 256), -jnp.inf, jnp.float32)
        sacc[...] = jnp.zeros((B, 256), jnp.float32)
        cacc[...] = jnp.zeros((B, 1), jnp.float32)

    bvec = batch_ref[...]
    h = h_ref[...]
    seg = jax.lax.broadcasted_iota(jnp.int32, (ROW_BLK, B), 1)
    onehot = (bvec == seg).astype(jnp.float32)
    sacc[...] += jax.lax.dot_general(
        onehot, h, (((0,), (0,)), ((), ())),
        preferred_element_type=jnp.float32)
    cacc[...] += jnp.sum(onehot, axis=0).reshape(B, 1)

    def seg_max(s, _):
        row = jnp.max(jnp.where(bvec == s, h, -jnp.inf), axis=0,
                      keepdims=True)
        macc[pl.ds(s, 1), :] = jnp.maximum(macc[pl.ds(s, 1), :], row)
        return 0

    lax.fori_loop(0, B, seg_max, 0)

    @pl.when(i == nsteps - 1)
    def _():
        gmean = sacc[...] / jnp.maximum(cacc[...], 1.0)
        g = jnp.concatenate([macc[...], gmean], axis=1)
        z = jnp.maximum(
            jnp.dot(g, fw1_ref[...], preferred_element_type=jnp.float32)
            + fb1_ref[...], 0.0)
        z = jnp.maximum(
            jnp.dot(z, fw2_ref[...], preferred_element_type=jnp.float32)
            + fb2_ref[...], 0.0)
        out_ref[...] = (
            jnp.dot(z, fw3_ref[...], preferred_element_type=jnp.float32)
            + fb3_ref[...])


def _pool_head_call(h, batch, fw1, fb1, fw2, fb2, fw3, fb3):
    grid = (N // ROW_BLK,)
    return pl.pallas_call(
        _pool_head_body,
        grid=grid,
        in_specs=[
            pl.BlockSpec((ROW_BLK, 256), lambda i: (i, 0)),
            pl.BlockSpec((ROW_BLK, 1), lambda i: (i, 0)),
            pl.BlockSpec((512, 512), lambda i: (0, 0)),
            pl.BlockSpec((1, 512), lambda i: (0, 0)),
            pl.BlockSpec((512, 256), lambda i: (0, 0)),
            pl.BlockSpec((1, 256), lambda i: (0, 0)),
            pl.BlockSpec((256, 1), lambda i: (0, 0)),
            pl.BlockSpec((1, 1), lambda i: (0, 0)),
        ],
        out_specs=pl.BlockSpec((B, 1), lambda i: (0, 0)),
        out_shape=jax.ShapeDtypeStruct((B, 1), jnp.float32),
        scratch_shapes=[
            pltpu.VMEM((B, 256), jnp.float32),
            pltpu.VMEM((B, 256), jnp.float32),
            pltpu.VMEM((B, 1), jnp.float32),
        ],
    )(h, batch.reshape(N, 1), fw1, fb1.reshape(1, -1), fw2,
      fb2.reshape(1, -1), fw3, fb3.reshape(1, -1))


def kernel(x, edge_index, batch, W1, b1, W2, b2, W3, b3, W4, b4,
           fw1, fb1, fw2, fb2, fw3, fb3):
    dst, src = lax.sort([edge_index[1], edge_index[0]], num_keys=1)
    src2d = src.reshape(EROWS, K)
    srcb2d = (src + N).reshape(EROWS, K)
    dst2d = dst.reshape(EROWS, K)
    zrows = jnp.zeros((ZR, DH), jnp.float32)

    deg = jax.ops.segment_sum(jnp.ones_like(dst, dtype=jnp.float32), dst,
                              num_segments=N)
    dinv, y = _pre_call(deg.reshape(N, 1), x)

    W1p = jnp.pad(W1, ((0, DH - 90), (0, 0)))

    def prop_narrow(y_flat):
        return _prop_sc(y_flat, src2d, src2d, dst2d, zrows, True)

    def prop_wide(y_split):
        return _prop_sc(y_split.reshape(2 * N, DH), src2d, srcb2d, dst2d,
                        zrows, False)

    y = _layer_call(prop_narrow(y), y, dinv, W1p, b1,
                    s_mode="sum", y_split=False, out_mode="flat")
    y = _layer_call(prop_narrow(y), y, dinv, W2, b2,
                    s_mode="sum", y_split=False, out_mode="split")
    y = _layer_call(prop_wide(y), y, dinv, W3, b3,
                    s_mode="concat", y_split=True, out_mode="split")
    h = _layer_call(prop_wide(y), y, dinv, W4, b4,
                    s_mode="concat", y_split=True, out_mode="plain")

    return _pool_head_call(h, batch, fw1, fb1, fw2, fb2, fw3, fb3)


# speedup vs baseline: 1.3135x; 1.3135x over previous
import functools

import jax
import jax.numpy as jnp
from jax import lax
from jax.experimental import pallas as pl
from jax.experimental.pallas import tpu as pltpu
from jax.experimental.pallas import tpu_sc as plsc

N = 10000
E = 640000
B = 64
ROW_BLK = 1000

NTILES = 16
K = 50
SUP = 40
EROWS = E // K
ZR = 624
TAIL = N - NTILES * ZR
DH = 128
RING = 5
LOOK = 2


def _prop_body(edge_split, tab, src_a, src_b, dst2d, zrows, out,
               acc, sbuf, dbuf, gbuf, gsem, ssem):
    cid = lax.axis_index("c")
    sid = lax.axis_index("s")

    pltpu.sync_copy(zrows, acc.at[pl.ds(sid * ZR, ZR)])

    @pl.when(sid == NTILES - 1)
    def _():
        pltpu.sync_copy(zrows.at[pl.ds(0, TAIL)],
                        acc.at[pl.ds(NTILES * ZR, TAIL)])

    plsc.subcore_barrier()

    if edge_split:
        cpt = EROWS // (2 * NTILES)
        row0 = (cid * NTILES + sid) * cpt
    else:
        cpt = EROWS // NTILES
        row0 = sid * cpt
    sups = cpt // SUP

    def sup_body(s, _):
        r0 = row0 + s * SUP

        @pl.when(cid == 0)
        def _():
            pltpu.sync_copy(src_a.at[pl.ds(r0, SUP)], sbuf)

        @pl.when(cid == 1)
        def _():
            pltpu.sync_copy(src_b.at[pl.ds(r0, SUP)], sbuf)

        pltpu.sync_copy(dst2d.at[pl.ds(r0, SUP)], dbuf)

        hg = [None] * RING
        hs = [None] * RING
        for j in range(LOOK):
            hg[j] = pltpu.async_copy(tab.at[sbuf.at[j]], gbuf.at[j],
                                     gsem.at[j])
        for j in range(SUP):
            r = j % RING
            jn = j + LOOK
            if jn < SUP:
                rn = jn % RING
                if hs[rn] is not None:
                    hs[rn].wait()
                hg[rn] = pltpu.async_copy(tab.at[sbuf.at[jn]], gbuf.at[rn],
                                          gsem.at[rn])
            hg[r].wait()
            hs[r] = pltpu.async_copy(gbuf.at[r], acc.at[dbuf.at[j]],
                                     ssem.at[r], add=True)
        for r in range(RING):
            if hs[r] is not None:
                hs[r].wait()
        return 0

    lax.fori_loop(0, sups, sup_body, 0)

    plsc.subcore_barrier()
    rows = pl.ds(sid * ZR, ZR)
    tail = pl.ds(NTILES * ZR, TAIL)

    @pl.when(cid == 0)
    def _():
        pltpu.sync_copy(acc.at[rows], out.at[0].at[rows])

        @pl.when(sid == NTILES - 1)
        def _():
            pltpu.sync_copy(acc.at[tail], out.at[0].at[tail])

    @pl.when(cid == 1)
    def _():
        pltpu.sync_copy(acc.at[rows], out.at[1].at[rows])

        @pl.when(sid == NTILES - 1)
        def _():
            pltpu.sync_copy(acc.at[tail], out.at[1].at[tail])


def _prop_sc(tab, src_a, src_b, dst2d, zrows, edge_split):
    mesh = plsc.VectorSubcoreMesh(core_axis_name="c", subcore_axis_name="s",
                                  num_cores=2, num_subcores=NTILES)
    return pl.kernel(
        functools.partial(_prop_body, edge_split),
        out_type=jax.ShapeDtypeStruct((2, N, DH), jnp.float32),
        mesh=mesh,
        scratch_types=[
            pltpu.VMEM_SHARED((N, DH), jnp.float32),
            pltpu.VMEM((SUP, K), jnp.int32),
            pltpu.VMEM((SUP, K), jnp.int32),
            pltpu.VMEM((RING, K, DH), jnp.float32),
            pltpu.SemaphoreType.DMA((RING,)),
            pltpu.SemaphoreType.DMA((RING,)),
        ],
    )(tab, src_a, src_b, dst2d, zrows)


def _pre_body(deg_ref, x_ref, dinv_ref, y_ref):
    deg = deg_ref[...]
    dinv = lax.rsqrt(deg + 1.0)
    dinv_ref[...] = dinv
    y = dinv * x_ref[...]
    y_ref[...] = jnp.concatenate(
        [y, jnp.zeros((y.shape[0], DH - 90), jnp.float32)], axis=1)


def _pre_call(deg, x):
    grid = (N // ROW_BLK,)
    return pl.pallas_call(
        _pre_body,
        grid=grid,
        in_specs=[
            pl.BlockSpec((ROW_BLK, 1), lambda i: (i, 0)),
            pl.BlockSpec((ROW_BLK, 90), lambda i: (i, 0)),
        ],
        out_specs=[
            pl.BlockSpec((ROW_BLK, 1), lambda i: (i, 0)),
            pl.BlockSpec((ROW_BLK, DH), lambda i: (i, 0)),
        ],
        out_shape=[
            jax.ShapeDtypeStruct((N, 1), jnp.float32),
            jax.ShapeDtypeStruct((N, DH), jnp.float32),
        ],
    )(deg, x)


def _layer_body(s_ref, y_ref, dinv_ref, w_ref, b_ref, out_ref, *,
                s_mode, y_split, out_mode):
    dinv = dinv_ref[...]
    if s_mode == "sum":
        s = s_ref[0] + s_ref[1]
    else:
        s = jnp.concatenate([s_ref[0], s_ref[1]], axis=1)
    if y_split:
        y = jnp.concatenate([y_ref[0], y_ref[1]], axis=1)
    else:
        y = y_ref[...]
    t = dinv * (s + y)
    h = jnp.maximum(
        jnp.dot(t, w_ref[...], preferred_element_type=jnp.float32)
        + b_ref[...],
        0.0,
    )
    if out_mode == "plain":
        out_ref[...] = h
    elif out_mode == "flat":
        yn = dinv * h
        out_ref[...] = yn
    else:
        yn = dinv * h
        out_ref[0] = yn[:, :DH]
        out_ref[1] = yn[:, DH:]


def _layer_call(s, y, dinv, w, b, *, s_mode, y_split, out_mode):
    din, dout = w.shape
    grid = (N // ROW_BLK,)
    if y_split:
        y_spec = pl.BlockSpec((2, ROW_BLK, DH), lambda i: (0, i, 0))
    else:
        y_spec = pl.BlockSpec((ROW_BLK, din), lambda i: (i, 0))
    if out_mode == "split":
        out_spec = pl.BlockSpec((2, ROW_BLK, DH), lambda i: (0, i, 0))
        out_shape = jax.ShapeDtypeStruct((2, N, DH), jnp.float32)
    else:
        out_spec = pl.BlockSpec((ROW_BLK, dout), lambda i: (i, 0))
        out_shape = jax.ShapeDtypeStruct((N, dout), jnp.float32)
    return pl.pallas_call(
        functools.partial(_layer_body, s_mode=s_mode, y_split=y_split,
                          out_mode=out_mode),
        grid=grid,
        in_specs=[
            pl.BlockSpec((2, ROW_BLK, DH), lambda i: (0, i, 0)),
            y_spec,
            pl.BlockSpec((ROW_BLK, 1), lambda i: (i, 0)),
            pl.BlockSpec((din, dout), lambda i: (0, 0)),
            pl.BlockSpec((1, dout), lambda i: (0, 0)),
        ],
        out_specs=out_spec,
        out_shape=out_shape,
    )(s, y, dinv, w, b.reshape(1, dout))


def _pool_head_body(h_ref, batch_ref, fw1_ref, fb1_ref, fw2_ref, fb2_ref,
                    fw3_ref, fb3_ref, out_ref, macc, sacc, cacc):
    i = pl.program_id(0)
    nsteps = pl.num_programs(0)

    @pl.when(i == 0)
    def _():
        macc[...] = jnp.full((B, 256), -jnp.inf, jnp.float32)
        sacc[...] = jnp.zeros((B, 256), jnp.float32)
        cacc[...] = jnp.zeros((B, 1), jnp.float32)

    bvec = batch_ref[...]
    h = h_ref[...]
    seg = jax.lax.broadcasted_iota(jnp.int32, (ROW_BLK, B), 1)
    onehot = (bvec == seg).astype(jnp.float32)
    sacc[...] += jax.lax.dot_general(
        onehot, h, (((0,), (0,)), ((), ())),
        preferred_element_type=jnp.float32)
    cacc[...] += jnp.sum(onehot, axis=0).reshape(B, 1)

    def seg_max(s, _):
        row = jnp.max(jnp.where(bvec == s, h, -jnp.inf), axis=0,
                      keepdims=True)
        macc[pl.ds(s, 1), :] = jnp.maximum(macc[pl.ds(s, 1), :], row)
        return 0

    lax.fori_loop(0, B, seg_max, 0)

    @pl.when(i == nsteps - 1)
    def _():
        gmean = sacc[...] / jnp.maximum(cacc[...], 1.0)
        g = jnp.concatenate([macc[...], gmean], axis=1)
        z = jnp.maximum(
            jnp.dot(g, fw1_ref[...], preferred_element_type=jnp.float32)
            + fb1_ref[...], 0.0)
        z = jnp.maximum(
            jnp.dot(z, fw2_ref[...], preferred_element_type=jnp.float32)
            + fb2_ref[...], 0.0)
        out_ref[...] = (
            jnp.dot(z, fw3_ref[...], preferred_element_type=jnp.float32)
            + fb3_ref[...])


def _pool_head_call(h, batch, fw1, fb1, fw2, fb2, fw3, fb3):
    grid = (N // ROW_BLK,)
    return pl.pallas_call(
        _pool_head_body,
        grid=grid,
        in_specs=[
            pl.BlockSpec((ROW_BLK, 256), lambda i: (i, 0)),
            pl.BlockSpec((ROW_BLK, 1), lambda i: (i, 0)),
            pl.BlockSpec((512, 512), lambda i: (0, 0)),
            pl.BlockSpec((1, 512), lambda i: (0, 0)),
            pl.BlockSpec((512, 256), lambda i: (0, 0)),
            pl.BlockSpec((1, 256), lambda i: (0, 0)),
            pl.BlockSpec((256, 1), lambda i: (0, 0)),
            pl.BlockSpec((1, 1), lambda i: (0, 0)),
        ],
        out_specs=pl.BlockSpec((B, 1), lambda i: (0, 0)),
        out_shape=jax.ShapeDtypeStruct((B, 1), jnp.float32),
        scratch_shapes=[
            pltpu.VMEM((B, 256), jnp.float32),
            pltpu.VMEM((B, 256), jnp.float32),
            pltpu.VMEM((B, 1), jnp.float32),
        ],
    )(h, batch.reshape(N, 1), fw1, fb1.reshape(1, -1), fw2,
      fb2.reshape(1, -1), fw3, fb3.reshape(1, -1))


def kernel(x, edge_index, batch, W1, b1, W2, b2, W3, b3, W4, b4,
           fw1, fb1, fw2, fb2, fw3, fb3):
    src = edge_index[0]
    dst = edge_index[1]
    src2d = src.reshape(EROWS, K)
    srcb2d = (src + N).reshape(EROWS, K)
    dst2d = dst.reshape(EROWS, K)
    zrows = jnp.zeros((ZR, DH), jnp.float32)

    deg = jax.ops.segment_sum(jnp.ones_like(dst, dtype=jnp.float32), dst,
                              num_segments=N)
    dinv, y = _pre_call(deg.reshape(N, 1), x)

    W1p = jnp.pad(W1, ((0, DH - 90), (0, 0)))

    def prop_narrow(y_flat):
        return _prop_sc(y_flat, src2d, src2d, dst2d, zrows, True)

    def prop_wide(y_split):
        return _prop_sc(y_split.reshape(2 * N, DH), src2d, srcb2d, dst2d,
                        zrows, False)

    y = _layer_call(prop_narrow(y), y, dinv, W1p, b1,
                    s_mode="sum", y_split=False, out_mode="flat")
    y = _layer_call(prop_narrow(y), y, dinv, W2, b2,
                    s_mode="sum", y_split=False, out_mode="split")
    y = _layer_call(prop_wide(y), y, dinv, W3, b3,
                    s_mode="concat", y_split=True, out_mode="split")
    h = _layer_call(prop_wide(y), y, dinv, W4, b4,
                    s_mode="concat", y_split=True, out_mode="plain")

    return _pool_head_call(h, batch, fw1, fb1, fw2, fb2, fw3, fb3)


# speedup vs baseline: 1.7053x; 1.2983x over previous
import functools

import jax
import jax.numpy as jnp
from jax import lax
from jax.experimental import pallas as pl
from jax.experimental.pallas import tpu as pltpu
from jax.experimental.pallas import tpu_sc as plsc

N = 10000
E = 640000
B = 64
ROW_BLK = 1000

NTILES = 16
K = 50
SUP = 40
EROWS = E // K
ZR = 624
TAIL = N - NTILES * ZR
DH = 128
RING = 5
LOOK = 2


def _prop_body(edge_split, tab, src_a, src_b, dst2d, zrows, out,
               acc, sbuf, dbuf, gbuf, gsem, ssem):
    cid = lax.axis_index("c")
    sid = lax.axis_index("s")

    pltpu.sync_copy(zrows, acc.at[pl.ds(sid * ZR, ZR)])

    @pl.when(sid == NTILES - 1)
    def _():
        pltpu.sync_copy(zrows.at[pl.ds(0, TAIL)],
                        acc.at[pl.ds(NTILES * ZR, TAIL)])

    plsc.subcore_barrier()

    if edge_split:
        cpt = EROWS // (2 * NTILES)
        row0 = (cid * NTILES + sid) * cpt
    else:
        cpt = EROWS // NTILES
        row0 = sid * cpt
    sups = cpt // SUP

    def sup_body(s, _):
        r0 = row0 + s * SUP

        @pl.when(cid == 0)
        def _():
            pltpu.sync_copy(src_a.at[pl.ds(r0, SUP)], sbuf)

        @pl.when(cid == 1)
        def _():
            pltpu.sync_copy(src_b.at[pl.ds(r0, SUP)], sbuf)

        pltpu.sync_copy(dst2d.at[pl.ds(r0, SUP)], dbuf)

        hg = [None] * RING
        hs = [None] * RING
        for j in range(LOOK):
            hg[j] = pltpu.async_copy(tab.at[sbuf.at[j]], gbuf.at[j],
                                     gsem.at[j])
        for j in range(SUP):
            r = j % RING
            jn = j + LOOK
            if jn < SUP:
                rn = jn % RING
                if hs[rn] is not None:
                    hs[rn].wait()
                hg[rn] = pltpu.async_copy(tab.at[sbuf.at[jn]], gbuf.at[rn],
                                          gsem.at[rn])
            hg[r].wait()
            hs[r] = pltpu.async_copy(gbuf.at[r], acc.at[dbuf.at[j]],
                                     ssem.at[r], add=True)
        for r in range(RING):
            if hs[r] is not None:
                hs[r].wait()
        return 0

    lax.fori_loop(0, sups, sup_body, 0)

    plsc.subcore_barrier()
    rows = pl.ds(sid * ZR, ZR)
    tail = pl.ds(NTILES * ZR, TAIL)

    @pl.when(cid == 0)
    def _():
        pltpu.sync_copy(acc.at[rows], out.at[0].at[rows])

        @pl.when(sid == NTILES - 1)
        def _():
            pltpu.sync_copy(acc.at[tail], out.at[0].at[tail])

    @pl.when(cid == 1)
    def _():
        pltpu.sync_copy(acc.at[rows], out.at[1].at[rows])

        @pl.when(sid == NTILES - 1)
        def _():
            pltpu.sync_copy(acc.at[tail], out.at[1].at[tail])


def _prop_sc(tab, src_a, src_b, dst2d, zrows, edge_split):
    mesh = plsc.VectorSubcoreMesh(core_axis_name="c", subcore_axis_name="s",
                                  num_cores=2, num_subcores=NTILES)
    return pl.kernel(
        functools.partial(_prop_body, edge_split),
        out_type=jax.ShapeDtypeStruct((2, N, DH), jnp.float32),
        mesh=mesh,
        scratch_types=[
            pltpu.VMEM_SHARED((N, DH), jnp.float32),
            pltpu.VMEM((SUP, K), jnp.int32),
            pltpu.VMEM((SUP, K), jnp.int32),
            pltpu.VMEM((RING, K, DH), jnp.float32),
            pltpu.SemaphoreType.DMA((RING,)),
            pltpu.SemaphoreType.DMA((RING,)),
        ],
    )(tab, src_a, src_b, dst2d, zrows)


def _deg_body(dst2d, ones_hbm, zrows, out, acc, dbuf, ones_v):
    cid = lax.axis_index("c")
    sid = lax.axis_index("s")

    pltpu.sync_copy(ones_hbm, ones_v)
    pltpu.sync_copy(zrows, acc.at[pl.ds(sid * ZR, ZR)])

    @pl.when(sid == NTILES - 1)
    def _():
        pltpu.sync_copy(zrows.at[pl.ds(0, TAIL)],
                        acc.at[pl.ds(NTILES * ZR, TAIL)])

    plsc.subcore_barrier()

    cpt = EROWS // (2 * NTILES)
    row0 = (cid * NTILES + sid) * cpt

    def sup_body(s, _):
        pltpu.sync_copy(dst2d.at[pl.ds(row0 + s * SUP, SUP)], dbuf)
        for j in range(SUP):
            pltpu.sync_copy(ones_v, acc.at[dbuf.at[j]], add=True)
        return 0

    lax.fori_loop(0, cpt // SUP, sup_body, 0)

    plsc.subcore_barrier()
    rows = pl.ds(sid * ZR, ZR)
    tail = pl.ds(NTILES * ZR, TAIL)

    @pl.when(cid == 0)
    def _():
        pltpu.sync_copy(acc.at[rows], out.at[0].at[rows])

        @pl.when(sid == NTILES - 1)
        def _():
            pltpu.sync_copy(acc.at[tail], out.at[0].at[tail])

    @pl.when(cid == 1)
    def _():
        pltpu.sync_copy(acc.at[rows], out.at[1].at[rows])

        @pl.when(sid == NTILES - 1)
        def _():
            pltpu.sync_copy(acc.at[tail], out.at[1].at[tail])


def _deg_sc(dst2d, ones_in, zrows):
    mesh = plsc.VectorSubcoreMesh(core_axis_name="c", subcore_axis_name="s",
                                  num_cores=2, num_subcores=NTILES)
    return pl.kernel(
        _deg_body,
        out_type=jax.ShapeDtypeStruct((2, N, DH), jnp.float32),
        mesh=mesh,
        scratch_types=[
            pltpu.VMEM_SHARED((N, DH), jnp.float32),
            pltpu.VMEM((SUP, K), jnp.int32),
            pltpu.VMEM((K, DH), jnp.float32),
        ],
    )(dst2d, ones_in, zrows)


def _pre_body(degp_ref, x_ref, dinv_ref, y_ref):
    deg = degp_ref[0][:, 0:1] + degp_ref[1][:, 0:1]
    dinv = lax.rsqrt(deg + 1.0)
    dinv_ref[...] = dinv
    y = dinv * x_ref[...]
    y_ref[...] = jnp.concatenate(
        [y, jnp.zeros((y.shape[0], DH - 90), jnp.float32)], axis=1)


def _pre_call(degp, x):
    grid = (N // ROW_BLK,)
    return pl.pallas_call(
        _pre_body,
        grid=grid,
        in_specs=[
            pl.BlockSpec((2, ROW_BLK, DH), lambda i: (0, i, 0)),
            pl.BlockSpec((ROW_BLK, 90), lambda i: (i, 0)),
        ],
        out_specs=[
            pl.BlockSpec((ROW_BLK, 1), lambda i: (i, 0)),
            pl.BlockSpec((ROW_BLK, DH), lambda i: (i, 0)),
        ],
        out_shape=[
            jax.ShapeDtypeStruct((N, 1), jnp.float32),
            jax.ShapeDtypeStruct((N, DH), jnp.float32),
        ],
    )(degp, x)


def _layer_body(s_ref, y_ref, dinv_ref, w_ref, b_ref, out_ref, *,
                s_mode, y_split, out_mode):
    dinv = dinv_ref[...]
    if s_mode == "sum":
        s = s_ref[0] + s_ref[1]
    else:
        s = jnp.concatenate([s_ref[0], s_ref[1]], axis=1)
    if y_split:
        y = jnp.concatenate([y_ref[0], y_ref[1]], axis=1)
    else:
        y = y_ref[...]
    t = dinv * (s + y)
    h = jnp.maximum(
        jnp.dot(t, w_ref[...], preferred_element_type=jnp.float32)
        + b_ref[...],
        0.0,
    )
    if out_mode == "plain":
        out_ref[...] = h
    elif out_mode == "flat":
        yn = dinv * h
        out_ref[...] = yn
    else:
        yn = dinv * h
        out_ref[0] = yn[:, :DH]
        out_ref[1] = yn[:, DH:]


def _layer_call(s, y, dinv, w, b, *, s_mode, y_split, out_mode):
    din, dout = w.shape
    grid = (N // ROW_BLK,)
    if y_split:
        y_spec = pl.BlockSpec((2, ROW_BLK, DH), lambda i: (0, i, 0))
    else:
        y_spec = pl.BlockSpec((ROW_BLK, din), lambda i: (i, 0))
    if out_mode == "split":
        out_spec = pl.BlockSpec((2, ROW_BLK, DH), lambda i: (0, i, 0))
        out_shape = jax.ShapeDtypeStruct((2, N, DH), jnp.float32)
    else:
        out_spec = pl.BlockSpec((ROW_BLK, dout), lambda i: (i, 0))
        out_shape = jax.ShapeDtypeStruct((N, dout), jnp.float32)
    return pl.pallas_call(
        functools.partial(_layer_body, s_mode=s_mode, y_split=y_split,
                          out_mode=out_mode),
        grid=grid,
        in_specs=[
            pl.BlockSpec((2, ROW_BLK, DH), lambda i: (0, i, 0)),
            y_spec,
            pl.BlockSpec((ROW_BLK, 1), lambda i: (i, 0)),
            pl.BlockSpec((din, dout), lambda i: (0, 0)),
            pl.BlockSpec((1, dout), lambda i: (0, 0)),
        ],
        out_specs=out_spec,
        out_shape=out_shape,
    )(s, y, dinv, w, b.reshape(1, dout))


def _pool_head_body(h_ref, batch_ref, fw1_ref, fb1_ref, fw2_ref, fb2_ref,
                    fw3_ref, fb3_ref, out_ref, macc, sacc, cacc):
    i = pl.program_id(0)
    nsteps = pl.num_programs(0)

    @pl.when(i == 0)
    def _():
        macc[...] = jnp.full((B, 256), -jnp.inf, jnp.float32)
        sacc[...] = jnp.zeros((B, 256), jnp.float32)
        cacc[...] = jnp.zeros((B, 1), jnp.float32)

    bvec = batch_ref[...]
    h = h_ref[...]
    seg = jax.lax.broadcasted_iota(jnp.int32, (ROW_BLK, B), 1)
    onehot = (bvec == seg).astype(jnp.float32)
    sacc[...] += jax.lax.dot_general(
        onehot, h, (((0,), (0,)), ((), ())),
        preferred_element_type=jnp.float32)
    cacc[...] += jnp.sum(onehot, axis=0).reshape(B, 1)

    def seg_max(s, _):
        row = jnp.max(jnp.where(bvec == s, h, -jnp.inf), axis=0,
                      keepdims=True)
        macc[pl.ds(s, 1), :] = jnp.maximum(macc[pl.ds(s, 1), :], row)
        return 0

    lax.fori_loop(0, B, seg_max, 0)

    @pl.when(i == nsteps - 1)
    def _():
        gmean = sacc[...] / jnp.maximum(cacc[...], 1.0)
        g = jnp.concatenate([macc[...], gmean], axis=1)
        z = jnp.maximum(
            jnp.dot(g, fw1_ref[...], preferred_element_type=jnp.float32)
            + fb1_ref[...], 0.0)
        z = jnp.maximum(
            jnp.dot(z, fw2_ref[...], preferred_element_type=jnp.float32)
            + fb2_ref[...], 0.0)
        out_ref[...] = (
            jnp.dot(z, fw3_ref[...], preferred_element_type=jnp.float32)
            + fb3_ref[...])


def _pool_head_call(h, batch, fw1, fb1, fw2, fb2, fw3, fb3):
    grid = (N // ROW_BLK,)
    return pl.pallas_call(
        _pool_head_body,
        grid=grid,
        in_specs=[
            pl.BlockSpec((ROW_BLK, 256), lambda i: (i, 0)),
            pl.BlockSpec((ROW_BLK, 1), lambda i: (i, 0)),
            pl.BlockSpec((512, 512), lambda i: (0, 0)),
            pl.BlockSpec((1, 512), lambda i: (0, 0)),
            pl.BlockSpec((512, 256), lambda i: (0, 0)),
            pl.BlockSpec((1, 256), lambda i: (0, 0)),
            pl.BlockSpec((256, 1), lambda i: (0, 0)),
            pl.BlockSpec((1, 1), lambda i: (0, 0)),
        ],
        out_specs=pl.BlockSpec((B, 1), lambda i: (0, 0)),
        out_shape=jax.ShapeDtypeStruct((B, 1), jnp.float32),
        scratch_shapes=[
            pltpu.VMEM((B, 256), jnp.float32),
            pltpu.VMEM((B, 256), jnp.float32),
            pltpu.VMEM((B, 1), jnp.float32),
        ],
    )(h, batch.reshape(N, 1), fw1, fb1.reshape(1, -1), fw2,
      fb2.reshape(1, -1), fw3, fb3.reshape(1, -1))


def kernel(x, edge_index, batch, W1, b1, W2, b2, W3, b3, W4, b4,
           fw1, fb1, fw2, fb2, fw3, fb3):
    src = edge_index[0]
    dst = edge_index[1]
    src2d = src.reshape(EROWS, K)
    srcb2d = (src + N).reshape(EROWS, K)
    dst2d = dst.reshape(EROWS, K)
    zrows = jnp.zeros((ZR, DH), jnp.float32)

    degp = _deg_sc(dst2d, jnp.ones((K, DH), jnp.float32), zrows)
    dinv, y = _pre_call(degp, x)

    W1p = jnp.pad(W1, ((0, DH - 90), (0, 0)))

    def prop_narrow(y_flat):
        return _prop_sc(y_flat, src2d, src2d, dst2d, zrows, True)

    def prop_wide(y_split):
        return _prop_sc(y_split.reshape(2 * N, DH), src2d, srcb2d, dst2d,
                        zrows, False)

    y = _layer_call(prop_narrow(y), y, dinv, W1p, b1,
                    s_mode="sum", y_split=False, out_mode="flat")
    y = _layer_call(prop_narrow(y), y, dinv, W2, b2,
                    s_mode="sum", y_split=False, out_mode="split")
    y = _layer_call(prop_wide(y), y, dinv, W3, b3,
                    s_mode="concat", y_split=True, out_mode="split")
    h = _layer_call(prop_wide(y), y, dinv, W4, b4,
                    s_mode="concat", y_split=True, out_mode="plain")

    return _pool_head_call(h, batch, fw1, fb1, fw2, fb2, fw3, fb3)


# speedup vs baseline: 1.7160x; 1.0063x over previous
import functools

import jax
import jax.numpy as jnp
from jax import lax
from jax.experimental import pallas as pl
from jax.experimental.pallas import tpu as pltpu
from jax.experimental.pallas import tpu_sc as plsc

N = 10000
E = 640000
B = 64
ROW_BLK = 1000

NTILES = 16
K = 50
SUP = 40
EROWS = E // K
ZR = 624
TAIL = N - NTILES * ZR
DH = 128
RING = 5
LOOK = 2


def _prop_body(edge_split, tab, src_a, src_b, dst2d, zrows, out,
               acc, sbuf, dbuf, gbuf, gsem, ssem):
    cid = lax.axis_index("c")
    sid = lax.axis_index("s")

    pltpu.sync_copy(zrows, acc.at[pl.ds(sid * ZR, ZR)])

    @pl.when(sid == NTILES - 1)
    def _():
        pltpu.sync_copy(zrows.at[pl.ds(0, TAIL)],
                        acc.at[pl.ds(NTILES * ZR, TAIL)])

    plsc.subcore_barrier()

    if edge_split:
        cpt = EROWS // (2 * NTILES)
        row0 = (cid * NTILES + sid) * cpt
    else:
        cpt = EROWS // NTILES
        row0 = sid * cpt
    sups = cpt // SUP

    def sup_body(s, _):
        r0 = row0 + s * SUP

        @pl.when(cid == 0)
        def _():
            pltpu.sync_copy(src_a.at[pl.ds(r0, SUP)], sbuf)

        @pl.when(cid == 1)
        def _():
            pltpu.sync_copy(src_b.at[pl.ds(r0, SUP)], sbuf)

        pltpu.sync_copy(dst2d.at[pl.ds(r0, SUP)], dbuf)

        hg = [None] * RING
        hs = [None] * RING
        for j in range(LOOK):
            hg[j] = pltpu.async_copy(tab.at[sbuf.at[j]], gbuf.at[j],
                                     gsem.at[j])
        for j in range(SUP):
            r = j % RING
            jn = j + LOOK
            if jn < SUP:
                rn = jn % RING
                if hs[rn] is not None:
                    hs[rn].wait()
                hg[rn] = pltpu.async_copy(tab.at[sbuf.at[jn]], gbuf.at[rn],
                                          gsem.at[rn])
            hg[r].wait()
            hs[r] = pltpu.async_copy(gbuf.at[r], acc.at[dbuf.at[j]],
                                     ssem.at[r], add=True)
        for r in range(RING):
            if hs[r] is not None:
                hs[r].wait()
        return 0

    lax.fori_loop(0, sups, sup_body, 0)

    plsc.subcore_barrier()
    rows = pl.ds(sid * ZR, ZR)
    tail = pl.ds(NTILES * ZR, TAIL)

    @pl.when(cid == 0)
    def _():
        pltpu.sync_copy(acc.at[rows], out.at[0].at[rows])

        @pl.when(sid == NTILES - 1)
        def _():
            pltpu.sync_copy(acc.at[tail], out.at[0].at[tail])

    @pl.when(cid == 1)
    def _():
        pltpu.sync_copy(acc.at[rows], out.at[1].at[rows])

        @pl.when(sid == NTILES - 1)
        def _():
            pltpu.sync_copy(acc.at[tail], out.at[1].at[tail])


def _prop_sc(tab, src_a, src_b, dst2d, zrows, edge_split):
    mesh = plsc.VectorSubcoreMesh(core_axis_name="c", subcore_axis_name="s",
                                  num_cores=2, num_subcores=NTILES)
    return pl.kernel(
        functools.partial(_prop_body, edge_split),
        out_type=jax.ShapeDtypeStruct((2, N, DH), jnp.float32),
        mesh=mesh,
        scratch_types=[
            pltpu.VMEM_SHARED((N, DH), jnp.float32),
            pltpu.VMEM((SUP, K), jnp.int32),
            pltpu.VMEM((SUP, K), jnp.int32),
            pltpu.VMEM((RING, K, DH), jnp.float32),
            pltpu.SemaphoreType.DMA((RING,)),
            pltpu.SemaphoreType.DMA((RING,)),
        ],
    )(tab, src_a, src_b, dst2d, zrows)


def _deg_body(dst2d, ones_hbm, zrows, out, acc, dbuf, ones_v, ssem):
    cid = lax.axis_index("c")
    sid = lax.axis_index("s")

    pltpu.sync_copy(ones_hbm, ones_v)
    pltpu.sync_copy(zrows, acc.at[pl.ds(sid * ZR, ZR)])

    @pl.when(sid == NTILES - 1)
    def _():
        pltpu.sync_copy(zrows.at[pl.ds(0, TAIL)],
                        acc.at[pl.ds(NTILES * ZR, TAIL)])

    plsc.subcore_barrier()

    cpt = EROWS // (2 * NTILES)
    row0 = (cid * NTILES + sid) * cpt

    def sup_body(s, _):
        pltpu.sync_copy(dst2d.at[pl.ds(row0 + s * SUP, SUP)], dbuf)
        hs = [pltpu.async_copy(ones_v, acc.at[dbuf.at[j]], ssem, add=True)
              for j in range(SUP)]
        for h in hs:
            h.wait()
        return 0

    lax.fori_loop(0, cpt // SUP, sup_body, 0)

    plsc.subcore_barrier()
    rows = pl.ds(sid * ZR, ZR)
    tail = pl.ds(NTILES * ZR, TAIL)

    @pl.when(cid == 0)
    def _():
        pltpu.sync_copy(acc.at[rows], out.at[0].at[rows])

        @pl.when(sid == NTILES - 1)
        def _():
            pltpu.sync_copy(acc.at[tail], out.at[0].at[tail])

    @pl.when(cid == 1)
    def _():
        pltpu.sync_copy(acc.at[rows], out.at[1].at[rows])

        @pl.when(sid == NTILES - 1)
        def _():
            pltpu.sync_copy(acc.at[tail], out.at[1].at[tail])


def _deg_sc(dst2d, ones_in, zrows):
    mesh = plsc.VectorSubcoreMesh(core_axis_name="c", subcore_axis_name="s",
                                  num_cores=2, num_subcores=NTILES)
    return pl.kernel(
        _deg_body,
        out_type=jax.ShapeDtypeStruct((2, N, DH), jnp.float32),
        mesh=mesh,
        scratch_types=[
            pltpu.VMEM_SHARED((N, DH), jnp.float32),
            pltpu.VMEM((SUP, K), jnp.int32),
            pltpu.VMEM((K, DH), jnp.float32),
            pltpu.SemaphoreType.DMA,
        ],
    )(dst2d, ones_in, zrows)


def _pre_body(degp_ref, x_ref, dinv_ref, y_ref):
    deg = degp_ref[0][:, 0:1] + degp_ref[1][:, 0:1]
    dinv = lax.rsqrt(deg + 1.0)
    dinv_ref[...] = dinv
    y = dinv * x_ref[...]
    y_ref[...] = jnp.concatenate(
        [y, jnp.zeros((y.shape[0], DH - 90), jnp.float32)], axis=1)


def _pre_call(degp, x):
    grid = (N // ROW_BLK,)
    return pl.pallas_call(
        _pre_body,
        grid=grid,
        in_specs=[
            pl.BlockSpec((2, ROW_BLK, DH), lambda i: (0, i, 0)),
            pl.BlockSpec((ROW_BLK, 90), lambda i: (i, 0)),
        ],
        out_specs=[
            pl.BlockSpec((ROW_BLK, 1), lambda i: (i, 0)),
            pl.BlockSpec((ROW_BLK, DH), lambda i: (i, 0)),
        ],
        out_shape=[
            jax.ShapeDtypeStruct((N, 1), jnp.float32),
            jax.ShapeDtypeStruct((N, DH), jnp.float32),
        ],
    )(degp, x)


def _layer_body(s_ref, y_ref, dinv_ref, w_ref, b_ref, out_ref, *,
                s_mode, y_split, out_mode):
    dinv = dinv_ref[...]
    if s_mode == "sum":
        s = s_ref[0] + s_ref[1]
    else:
        s = jnp.concatenate([s_ref[0], s_ref[1]], axis=1)
    if y_split:
        y = jnp.concatenate([y_ref[0], y_ref[1]], axis=1)
    else:
        y = y_ref[...]
    t = dinv * (s + y)
    h = jnp.maximum(
        jnp.dot(t, w_ref[...], preferred_element_type=jnp.float32)
        + b_ref[...],
        0.0,
    )
    if out_mode == "plain":
        out_ref[...] = h
    elif out_mode == "flat":
        yn = dinv * h
        out_ref[...] = yn
    else:
        yn = dinv * h
        out_ref[0] = yn[:, :DH]
        out_ref[1] = yn[:, DH:]


def _layer_call(s, y, dinv, w, b, *, s_mode, y_split, out_mode):
    din, dout = w.shape
    grid = (N // ROW_BLK,)
    if y_split:
        y_spec = pl.BlockSpec((2, ROW_BLK, DH), lambda i: (0, i, 0))
    else:
        y_spec = pl.BlockSpec((ROW_BLK, din), lambda i: (i, 0))
    if out_mode == "split":
        out_spec = pl.BlockSpec((2, ROW_BLK, DH), lambda i: (0, i, 0))
        out_shape = jax.ShapeDtypeStruct((2, N, DH), jnp.float32)
    else:
        out_spec = pl.BlockSpec((ROW_BLK, dout), lambda i: (i, 0))
        out_shape = jax.ShapeDtypeStruct((N, dout), jnp.float32)
    return pl.pallas_call(
        functools.partial(_layer_body, s_mode=s_mode, y_split=y_split,
                          out_mode=out_mode),
        grid=grid,
        in_specs=[
            pl.BlockSpec((2, ROW_BLK, DH), lambda i: (0, i, 0)),
            y_spec,
            pl.BlockSpec((ROW_BLK, 1), lambda i: (i, 0)),
            pl.BlockSpec((din, dout), lambda i: (0, 0)),
            pl.BlockSpec((1, dout), lambda i: (0, 0)),
        ],
        out_specs=out_spec,
        out_shape=out_shape,
    )(s, y, dinv, w, b.reshape(1, dout))


def _pool_head_body(h_ref, batch_ref, fw1_ref, fb1_ref, fw2_ref, fb2_ref,
                    fw3_ref, fb3_ref, out_ref, macc, sacc, cacc):
    i = pl.program_id(0)
    nsteps = pl.num_programs(0)

    @pl.when(i == 0)
    def _():
        macc[...] = jnp.full((B, 256), -jnp.inf, jnp.float32)
        sacc[...] = jnp.zeros((B, 256), jnp.float32)
        cacc[...] = jnp.zeros((B, 1), jnp.float32)

    bvec = batch_ref[...]
    h = h_ref[...]
    seg = jax.lax.broadcasted_iota(jnp.int32, (ROW_BLK, B), 1)
    onehot = (bvec == seg).astype(jnp.float32)
    sacc[...] += jax.lax.dot_general(
        onehot, h, (((0,), (0,)), ((), ())),
        preferred_element_type=jnp.float32)
    cacc[...] += jnp.sum(onehot, axis=0).reshape(B, 1)

    def seg_max(s, _):
        row = jnp.max(jnp.where(bvec == s, h, -jnp.inf), axis=0,
                      keepdims=True)
        macc[pl.ds(s, 1), :] = jnp.maximum(macc[pl.ds(s, 1), :], row)
        return 0

    lax.fori_loop(0, B, seg_max, 0)

    @pl.when(i == nsteps - 1)
    def _():
        gmean = sacc[...] / jnp.maximum(cacc[...], 1.0)
        g = jnp.concatenate([macc[...], gmean], axis=1)
        z = jnp.maximum(
            jnp.dot(g, fw1_ref[...], preferred_element_type=jnp.float32)
            + fb1_ref[...], 0.0)
        z = jnp.maximum(
            jnp.dot(z, fw2_ref[...], preferred_element_type=jnp.float32)
            + fb2_ref[...], 0.0)
        out_ref[...] = (
            jnp.dot(z, fw3_ref[...], preferred_element_type=jnp.float32)
            + fb3_ref[...])


def _pool_head_call(h, batch, fw1, fb1, fw2, fb2, fw3, fb3):
    grid = (N // ROW_BLK,)
    return pl.pallas_call(
        _pool_head_body,
        grid=grid,
        in_specs=[
            pl.BlockSpec((ROW_BLK, 256), lambda i: (i, 0)),
            pl.BlockSpec((ROW_BLK, 1), lambda i: (i, 0)),
            pl.BlockSpec((512, 512), lambda i: (0, 0)),
            pl.BlockSpec((1, 512), lambda i: (0, 0)),
            pl.BlockSpec((512, 256), lambda i: (0, 0)),
            pl.BlockSpec((1, 256), lambda i: (0, 0)),
            pl.BlockSpec((256, 1), lambda i: (0, 0)),
            pl.BlockSpec((1, 1), lambda i: (0, 0)),
        ],
        out_specs=pl.BlockSpec((B, 1), lambda i: (0, 0)),
        out_shape=jax.ShapeDtypeStruct((B, 1), jnp.float32),
        scratch_shapes=[
            pltpu.VMEM((B, 256), jnp.float32),
            pltpu.VMEM((B, 256), jnp.float32),
            pltpu.VMEM((B, 1), jnp.float32),
        ],
    )(h, batch.reshape(N, 1), fw1, fb1.reshape(1, -1), fw2,
      fb2.reshape(1, -1), fw3, fb3.reshape(1, -1))


def kernel(x, edge_index, batch, W1, b1, W2, b2, W3, b3, W4, b4,
           fw1, fb1, fw2, fb2, fw3, fb3):
    src = edge_index[0]
    dst = edge_index[1]
    src2d = src.reshape(EROWS, K)
    srcb2d = (src + N).reshape(EROWS, K)
    dst2d = dst.reshape(EROWS, K)
    zrows = jnp.zeros((ZR, DH), jnp.float32)

    degp = _deg_sc(dst2d, jnp.ones((K, DH), jnp.float32), zrows)
    dinv, y = _pre_call(degp, x)

    W1p = jnp.pad(W1, ((0, DH - 90), (0, 0)))

    def prop_narrow(y_flat):
        return _prop_sc(y_flat, src2d, src2d, dst2d, zrows, True)

    def prop_wide(y_split):
        return _prop_sc(y_split.reshape(2 * N, DH), src2d, srcb2d, dst2d,
                        zrows, False)

    y = _layer_call(prop_narrow(y), y, dinv, W1p, b1,
                    s_mode="sum", y_split=False, out_mode="flat")
    y = _layer_call(prop_narrow(y), y, dinv, W2, b2,
                    s_mode="sum", y_split=False, out_mode="split")
    y = _layer_call(prop_wide(y), y, dinv, W3, b3,
                    s_mode="concat", y_split=True, out_mode="split")
    h = _layer_call(prop_wide(y), y, dinv, W4, b4,
                    s_mode="concat", y_split=True, out_mode="plain")

    return _pool_head_call(h, batch, fw1, fb1, fw2, fb2, fw3, fb3)
